# baseline (speedup 1.0000x reference)
import jax
import jax.numpy as jnp
from jax import lax
from jax.experimental import pallas as pl
from jax.experimental.pallas import tpu as pltpu
from jax.experimental.pallas import tpu_sc as plsc

N = 10000
NP = 10240
E = 320000
D = 128
NC = 2
NS = 16
CB = 80
EPS = E // NC
EPT = EPS // NS
NCHUNK = EPT // CB
RPT = NP // NS
ZROWS = 128
DR = NP // D


def _prologue_body(h_ref, norm_ref, out_ref):
    h1 = h_ref[...] * norm_ref[...]
    pad = jnp.zeros((NP - N, D), jnp.float32)
    out_ref[...] = jnp.concatenate([h1, pad], axis=0)


def _prologue(h, norm):
    return pl.pallas_call(
        _prologue_body,
        out_shape=jax.ShapeDtypeStruct((NP, D), jnp.float32),
    )(h, norm)


def _sc_body(h1p, src_hbm, dst_hbm, csum_hbm, deg_hbm,
             src_a, src_b, src_c, dst_a, dst_b, dst_c,
             rows_a, rows_b, rows_c,
             deg_loc, ident, acc_sh, deg_sh,
             isem_a, isem_b, isem_c, gsem_a, gsem_b, gsem_c):
    c = lax.axis_index("c")
    s = lax.axis_index("s")
    ebase = c * EPS + s * EPT
    ones16 = jnp.ones((16,), jnp.float32)

    def fire_idx(j, src_v, dst_v, isem):
        off = ebase + j * CB
        pltpu.make_async_copy(
            src_hbm.at[pl.ds(off, CB)], src_v, isem).start()
        pltpu.make_async_copy(
            dst_hbm.at[pl.ds(off, CB)], dst_v, isem).start()

    def wait_idx(src_v, dst_v, isem):
        pltpu.make_async_copy(
            src_hbm.at[pl.ds(0, CB)], src_v, isem).wait()
        pltpu.make_async_copy(
            dst_hbm.at[pl.ds(0, CB)], dst_v, isem).wait()

    def fire_gather(src_v, rows_v, gsem):
        pltpu.make_async_copy(h1p.at[src_v], rows_v, gsem).start()

    def wait_gather(src_v, rows_v, gsem):
        pltpu.make_async_copy(h1p.at[src_v], rows_v, gsem).wait()

    def histogram(dst_v):
        for i in range(CB // 16):
            d16 = dst_v[pl.ds(i * 16, 16)]
            hi = lax.shift_right_logical(d16, 7)
            lo = lax.bitwise_and(d16, 127)
            counts, last = plsc.scan_count(d16)
            plsc.addupdate_scatter(
                deg_loc, [hi, lo], counts.astype(jnp.float32), mask=last)

    slots = ((src_a, dst_a, rows_a, isem_a, gsem_a),
             (src_b, dst_b, rows_b, isem_b, gsem_b),
             (src_c, dst_c, rows_c, isem_c, gsem_c))

    def start(j, slot):
        src_v, dst_v, rows_v, isem, gsem = slot
        wait_idx(src_v, dst_v, isem)
        fire_gather(src_v, rows_v, gsem)

    def finish(j, slot):
        src_v, dst_v, rows_v, isem, gsem = slot
        histogram(dst_v)
        wait_gather(src_v, rows_v, gsem)
        pltpu.sync_copy(rows_v, acc_sh.at[dst_v], add=True)
        fire_idx(lax.rem(j + 3, NCHUNK), src_v, dst_v, isem)

    for j in range(3):
        fire_idx(j, slots[j][0], slots[j][1], slots[j][3])
    start(0, slots[0])
    start(1, slots[1])

    @pl.loop(0, CB)
    def _(i):
        @pl.loop(0, D // 16)
        def _(j):
            rows_c[i, pl.ds(j * 16, 16)] = jnp.zeros((16,), jnp.float32)

    @pl.loop(0, RPT // CB)
    def _(k):
        pltpu.sync_copy(rows_c, acc_sh.at[pl.ds(s * RPT + k * CB, CB)])

    @pl.when(s == 0)
    def _():
        pltpu.sync_copy(rows_c, deg_sh)

    @pl.loop(0, DR)
    def _(i):
        @pl.loop(0, D // 16)
        def _(j):
            deg_loc[i, pl.ds(j * 16, 16)] = jnp.zeros((16,), jnp.float32)

    @pl.loop(0, DR // 16)
    def _(i):
        ident[pl.ds(i * 16, 16)] = lax.iota(jnp.int32, 16) + i * 16

    plsc.subcore_barrier()

    @pl.loop(0, (NCHUNK - 2) // 3)
    def _(t):
        for k in range(3):
            j = 3 * t + k
            finish(j, slots[k])
            start(j + 2, slots[(k + 2) % 3])

    finish(NCHUNK - 2, slots[0])
    finish(NCHUNK - 1, slots[1])
    for k in range(3):
        wait_idx(slots[k][0], slots[k][1], slots[k][3])

    pltpu.sync_copy(deg_loc, deg_sh.at[ident], add=True)

    plsc.subcore_barrier()
    row0 = s * RPT
    pltpu.sync_copy(acc_sh.at[pl.ds(row0, RPT)],
                    csum_hbm.at[pl.ds(c * NP + row0, RPT)])

    @pl.when(s < 5)
    def _():
        pltpu.sync_copy(deg_sh.at[pl.ds(s * 16, 16)],
                        deg_hbm.at[pl.ds(c * DR + s * 16, 16)])


def _sc_segment(h1p, src, dst):
    mesh = plsc.VectorSubcoreMesh(
        core_axis_name="c", subcore_axis_name="s",
        num_cores=NC, num_subcores=NS)
    return pl.kernel(
        _sc_body,
        out_type=(jax.ShapeDtypeStruct((2 * NP, D), jnp.float32),
                  jax.ShapeDtypeStruct((2 * DR, D), jnp.float32)),
        mesh=mesh,
        compiler_params=pltpu.CompilerParams(needs_layout_passes=False),
        scratch_types=[
            pltpu.VMEM((CB,), jnp.int32),
            pltpu.VMEM((CB,), jnp.int32),
            pltpu.VMEM((CB,), jnp.int32),
            pltpu.VMEM((CB,), jnp.int32),
            pltpu.VMEM((CB,), jnp.int32),
            pltpu.VMEM((CB,), jnp.int32),
            pltpu.VMEM((CB, D), jnp.float32),
            pltpu.VMEM((CB, D), jnp.float32),
            pltpu.VMEM((CB, D), jnp.float32),
            pltpu.VMEM((DR, D), jnp.float32),
            pltpu.VMEM((DR,), jnp.int32),
            pltpu.VMEM_SHARED((NP, D), jnp.float32),
            pltpu.VMEM_SHARED((DR, D), jnp.float32),
            pltpu.SemaphoreType.DMA,
            pltpu.SemaphoreType.DMA,
            pltpu.SemaphoreType.DMA,
            pltpu.SemaphoreType.DMA,
            pltpu.SemaphoreType.DMA,
            pltpu.SemaphoreType.DMA,
        ],
    )(h1p, src, dst)


def _epilogue_body(h_ref, norm_ref, csum_ref, deg_ref, gamma_ref, beta_ref,
                   out_ref):
    norm = norm_ref[...]
    h1 = h_ref[...] * norm
    rdeg = pl.reciprocal(
        jnp.maximum(deg_ref[:N, 0:1] + deg_ref[:N, 1:2], 1.0), approx=False)
    cmean = (csum_ref[:N, :] + csum_ref[NP:NP + N, :]) * rdeg
    ssq = (jnp.sum(h1 * h1, axis=1, keepdims=True)
           + jnp.sum(cmean * cmean, axis=1, keepdims=True))
    scale = norm * jax.lax.rsqrt(jnp.maximum(ssq, 1e-24))
    h2 = jnp.concatenate([h1, cmean], axis=1) * scale
    h2sq = h2 * h2
    mean = jnp.mean(h2, axis=0, keepdims=True)
    var = jnp.maximum(jnp.mean(h2sq, axis=0, keepdims=True) - mean * mean,
                      0.0)
    inv = jax.lax.rsqrt(var + 1e-5)
    a = inv * gamma_ref[...]
    b = beta_ref[...] - mean * a
    out_ref[...] = h2 * a + b


def _epilogue(h, norm, csum, deg, gamma, beta):
    return pl.pallas_call(
        _epilogue_body,
        out_shape=jax.ShapeDtypeStruct((N, 2 * D), jnp.float32),
    )(h, norm, csum, jnp.transpose(deg.reshape(2, NP)),
      gamma.reshape(1, 2 * D), beta.reshape(1, 2 * D))


def kernel(h, edge_index, norm, gamma, beta):
    h1p = _prologue(h, norm)
    csum, deg = _sc_segment(h1p, edge_index[0], edge_index[1])
    return _epilogue(h, norm, csum, deg, gamma, beta)


# speedup vs baseline: 1.1554x; 1.1554x over previous
import jax
import jax.numpy as jnp
from jax import lax
from jax.experimental import pallas as pl
from jax.experimental.pallas import tpu as pltpu
from jax.experimental.pallas import tpu_sc as plsc

N = 10000
NP = 10240
E = 320000
D = 128
NC = 2
NS = 16
CB = 80
EPS = E // NC
EPT = EPS // NS
NCHUNK = EPT // CB
RPT = NP // NS
ZROWS = 128
DR = NP // D


def _prologue_body(h_ref, norm_ref, out_ref):
    h1 = h_ref[...] * norm_ref[...]
    pad = jnp.zeros((NP - N, D), jnp.float32)
    out_ref[...] = jnp.concatenate([h1, pad], axis=0)


def _prologue(h, norm):
    return pl.pallas_call(
        _prologue_body,
        out_shape=jax.ShapeDtypeStruct((NP, D), jnp.float32),
    )(h, norm)


def _sc_body(h1p, edges_hbm, csum_hbm, deg_hbm,
             src_a, src_b, src_c, dst_a, dst_b, dst_c,
             sdst_a, sdst_b, sdst_c,
             rows_a, rows_b, rows_c,
             deg_loc, ident, acc_sh, deg_sh,
             isem_a, isem_b, isem_c, gsem_a, gsem_b, gsem_c,
             ssem_a, ssem_b, ssem_c):
    c = lax.axis_index("c")
    s = lax.axis_index("s")
    ebase = c * EPS + s * EPT
    ones16 = jnp.ones((16,), jnp.float32)

    def fire_idx(j, src_v, dst_v, isem):
        off = ebase + j * CB
        pltpu.make_async_copy(
            edges_hbm.at[pl.ds(off, CB)], src_v, isem).start()
        pltpu.make_async_copy(
            edges_hbm.at[pl.ds(E + off, CB)], dst_v, isem).start()

    def wait_idx(src_v, dst_v, isem):
        pltpu.make_async_copy(
            edges_hbm.at[pl.ds(0, CB)], src_v, isem).wait()
        pltpu.make_async_copy(
            edges_hbm.at[pl.ds(0, CB)], dst_v, isem).wait()

    def fire_gather(src_v, rows_v, gsem):
        pltpu.make_async_copy(h1p.at[src_v], rows_v, gsem).start()

    def wait_gather(src_v, rows_v, gsem):
        pltpu.make_async_copy(h1p.at[src_v], rows_v, gsem).wait()

    def histogram(dst_v):
        for i in range(CB // 16):
            d16 = dst_v[pl.ds(i * 16, 16)]
            hi = lax.shift_right_logical(d16, 7)
            lo = lax.bitwise_and(d16, 127)
            counts, last = plsc.scan_count(d16)
            plsc.addupdate_scatter(
                deg_loc, [hi, lo], counts.astype(jnp.float32), mask=last)

    slots = ((src_a, dst_a, sdst_a, rows_a, isem_a, gsem_a, ssem_a),
             (src_b, dst_b, sdst_b, rows_b, isem_b, gsem_b, ssem_b),
             (src_c, dst_c, sdst_c, rows_c, isem_c, gsem_c, ssem_c))

    def wait_scatter(slot):
        _, _, sdst_v, rows_v, _, _, ssem = slot
        pltpu.make_async_copy(rows_v, acc_sh.at[sdst_v], ssem).wait()

    def start(j, slot):
        src_v, dst_v, sdst_v, rows_v, isem, gsem, ssem = slot
        wait_idx(src_v, dst_v, isem)
        wait_scatter(slot)
        fire_gather(src_v, rows_v, gsem)

    def finish(j, slot):
        src_v, dst_v, sdst_v, rows_v, isem, gsem, ssem = slot
        histogram(dst_v)
        wait_gather(src_v, rows_v, gsem)
        @pl.loop(0, CB // 16)
        def _(i):
            sdst_v[pl.ds(i * 16, 16)] = dst_v[pl.ds(i * 16, 16)]
        pltpu.async_copy(rows_v, acc_sh.at[sdst_v], ssem, add=True)
        fire_idx(lax.rem(j + 3, NCHUNK), src_v, dst_v, isem)

    for j in range(3):
        fire_idx(j, slots[j][0], slots[j][1], slots[j][4])

    for rows_v in (rows_a, rows_b, rows_c):
        @pl.loop(0, CB)
        def _(i):
            @pl.loop(0, D // 16)
            def _(j):
                rows_v[i, pl.ds(j * 16, 16)] = jnp.zeros((16,), jnp.float32)

    @pl.loop(0, RPT // CB)
    def _(k):
        pltpu.sync_copy(rows_c, acc_sh.at[pl.ds(s * RPT + k * CB, CB)])

    @pl.when(s == 0)
    def _():
        pltpu.sync_copy(rows_c, deg_sh)

    @pl.loop(0, DR)
    def _(i):
        @pl.loop(0, D // 16)
        def _(j):
            deg_loc[i, pl.ds(j * 16, 16)] = jnp.zeros((16,), jnp.float32)

    @pl.loop(0, DR // 16)
    def _(i):
        ident[pl.ds(i * 16, 16)] = lax.iota(jnp.int32, 16) + i * 16

    for slot in slots:
        _, _, sdst_v, rows_v, _, _, ssem = slot

        @pl.loop(0, DR // 16)
        def _(i):
            sdst_v[pl.ds(i * 16, 16)] = ident[pl.ds(i * 16, 16)]
        pltpu.async_copy(rows_v, acc_sh.at[sdst_v], ssem, add=True)

    start(0, slots[0])
    start(1, slots[1])

    plsc.subcore_barrier()

    @pl.loop(0, (NCHUNK - 2) // 3)
    def _(t):
        for k in range(3):
            j = 3 * t + k
            finish(j, slots[k])
            start(j + 2, slots[(k + 2) % 3])

    finish(NCHUNK - 2, slots[0])
    finish(NCHUNK - 1, slots[1])
    for k in range(3):
        wait_idx(slots[k][0], slots[k][1], slots[k][4])
        wait_scatter(slots[k])

    pltpu.sync_copy(deg_loc, deg_sh.at[ident], add=True)

    plsc.subcore_barrier()
    row0 = s * RPT
    pltpu.sync_copy(acc_sh.at[pl.ds(row0, RPT)],
                    csum_hbm.at[pl.ds(c * NP + row0, RPT)])

    @pl.when(s < 5)
    def _():
        pltpu.sync_copy(deg_sh.at[pl.ds(s * 16, 16)],
                        deg_hbm.at[pl.ds(c * DR + s * 16, 16)])


def _sc_segment(h1p, edge_index):
    mesh = plsc.VectorSubcoreMesh(
        core_axis_name="c", subcore_axis_name="s",
        num_cores=NC, num_subcores=NS)
    return pl.kernel(
        _sc_body,
        out_type=(jax.ShapeDtypeStruct((2 * NP, D), jnp.float32),
                  jax.ShapeDtypeStruct((2 * DR, D), jnp.float32)),
        mesh=mesh,
        compiler_params=pltpu.CompilerParams(needs_layout_passes=False),
        scratch_types=[
            pltpu.VMEM((CB,), jnp.int32),
            pltpu.VMEM((CB,), jnp.int32),
            pltpu.VMEM((CB,), jnp.int32),
            pltpu.VMEM((CB,), jnp.int32),
            pltpu.VMEM((CB,), jnp.int32),
            pltpu.VMEM((CB,), jnp.int32),
            pltpu.VMEM((CB,), jnp.int32),
            pltpu.VMEM((CB,), jnp.int32),
            pltpu.VMEM((CB,), jnp.int32),
            pltpu.VMEM((CB, D), jnp.float32),
            pltpu.VMEM((CB, D), jnp.float32),
            pltpu.VMEM((CB, D), jnp.float32),
            pltpu.VMEM((DR, D), jnp.float32),
            pltpu.VMEM((DR,), jnp.int32),
            pltpu.VMEM_SHARED((NP, D), jnp.float32),
            pltpu.VMEM_SHARED((DR, D), jnp.float32),
            pltpu.SemaphoreType.DMA,
            pltpu.SemaphoreType.DMA,
            pltpu.SemaphoreType.DMA,
            pltpu.SemaphoreType.DMA,
            pltpu.SemaphoreType.DMA,
            pltpu.SemaphoreType.DMA,
            pltpu.SemaphoreType.DMA,
            pltpu.SemaphoreType.DMA,
            pltpu.SemaphoreType.DMA,
        ],
    )(h1p, edge_index.reshape(2 * E))


def _epilogue_body(h_ref, norm_ref, csum_ref, deg_ref, gamma_ref, beta_ref,
                   out_ref):
    norm = norm_ref[...]
    h1 = h_ref[...] * norm
    rdeg = pl.reciprocal(
        jnp.maximum(deg_ref[:N, 0:1] + deg_ref[:N, 1:2], 1.0), approx=False)
    cmean = (csum_ref[:N, :] + csum_ref[NP:NP + N, :]) * rdeg
    ssq = (jnp.sum(h1 * h1, axis=1, keepdims=True)
           + jnp.sum(cmean * cmean, axis=1, keepdims=True))
    scale = norm * jax.lax.rsqrt(jnp.maximum(ssq, 1e-24))
    h2 = jnp.concatenate([h1, cmean], axis=1) * scale
    h2sq = h2 * h2
    mean = jnp.mean(h2, axis=0, keepdims=True)
    var = jnp.maximum(jnp.mean(h2sq, axis=0, keepdims=True) - mean * mean,
                      0.0)
    inv = jax.lax.rsqrt(var + 1e-5)
    a = inv * gamma_ref[...]
    b = beta_ref[...] - mean * a
    out_ref[...] = h2 * a + b


def _epilogue(h, norm, csum, deg, gamma, beta):
    return pl.pallas_call(
        _epilogue_body,
        out_shape=jax.ShapeDtypeStruct((N, 2 * D), jnp.float32),
    )(h, norm, csum, jnp.transpose(deg.reshape(2, NP)),
      gamma.reshape(1, 2 * D), beta.reshape(1, 2 * D))


def kernel(h, edge_index, norm, gamma, beta):
    h1p = _prologue(h, norm)
    csum, deg = _sc_segment(h1p, edge_index)
    return _epilogue(h, norm, csum, deg, gamma, beta)


# speedup vs baseline: 1.1558x; 1.0003x over previous
import jax
import jax.numpy as jnp
from jax import lax
from jax.experimental import pallas as pl
from jax.experimental.pallas import tpu as pltpu
from jax.experimental.pallas import tpu_sc as plsc

N = 10000
NP = 10240
E = 320000
D = 128
NC = 2
NS = 16
CB = 80
EPS = E // NC
EPT = EPS // NS
NCHUNK = EPT // CB
RPT = NP // NS
ZROWS = 128
DR = NP // D


def _prologue_body(h_ref, norm_ref, out_ref):
    h1 = h_ref[...] * norm_ref[...]
    pad = jnp.zeros((NP - N, D), jnp.float32)
    out_ref[...] = jnp.concatenate([h1, pad], axis=0)


def _prologue(h, norm):
    return pl.pallas_call(
        _prologue_body,
        out_shape=jax.ShapeDtypeStruct((NP, D), jnp.float32),
    )(h, norm)


def _sc_body(h1p, edges_hbm, csum_hbm, deg_hbm,
             src_a, src_b, src_c, dst_a, dst_b, dst_c,
             sdst_a, sdst_b, sdst_c,
             rows_a, rows_b, rows_c,
             deg_loc, ident, acc_sh, deg_sh,
             isem_a, isem_b, isem_c, gsem_a, gsem_b, gsem_c,
             ssem_a, ssem_b, ssem_c):
    c = lax.axis_index("c")
    s = lax.axis_index("s")
    ebase = c * EPS + s * EPT
    ones16 = jnp.ones((16,), jnp.float32)

    def fire_idx(j, src_v, dst_v, isem):
        off = ebase + j * CB
        pltpu.make_async_copy(
            edges_hbm.at[pl.ds(off, CB)], src_v, isem).start()
        pltpu.make_async_copy(
            edges_hbm.at[pl.ds(E + off, CB)], dst_v, isem).start()

    def wait_idx(src_v, dst_v, isem):
        pltpu.make_async_copy(
            edges_hbm.at[pl.ds(0, CB)], src_v, isem).wait()
        pltpu.make_async_copy(
            edges_hbm.at[pl.ds(0, CB)], dst_v, isem).wait()

    def fire_gather(src_v, rows_v, gsem):
        pltpu.make_async_copy(h1p.at[src_v], rows_v, gsem).start()

    def wait_gather(src_v, rows_v, gsem):
        pltpu.make_async_copy(h1p.at[src_v], rows_v, gsem).wait()

    def histogram(dst_v):
        for i in range(CB // 16):
            d16 = dst_v[pl.ds(i * 16, 16)]
            hi = lax.shift_right_logical(d16, 7)
            lo = lax.bitwise_and(d16, 127)
            counts, last = plsc.scan_count(d16)
            plsc.addupdate_scatter(
                deg_loc, [hi, lo], counts.astype(jnp.float32), mask=last)

    slots = ((src_a, dst_a, sdst_a, rows_a, isem_a, gsem_a, ssem_a),
             (src_b, dst_b, sdst_b, rows_b, isem_b, gsem_b, ssem_b),
             (src_c, dst_c, sdst_c, rows_c, isem_c, gsem_c, ssem_c))

    def wait_scatter(slot):
        _, _, sdst_v, rows_v, _, _, ssem = slot
        pltpu.make_async_copy(rows_v, acc_sh.at[sdst_v], ssem).wait()

    def start(j, slot):
        src_v, dst_v, sdst_v, rows_v, isem, gsem, ssem = slot
        wait_idx(src_v, dst_v, isem)
        wait_scatter(slot)
        fire_gather(src_v, rows_v, gsem)

    def finish(j, slot):
        src_v, dst_v, sdst_v, rows_v, isem, gsem, ssem = slot
        histogram(dst_v)
        wait_gather(src_v, rows_v, gsem)
        @pl.loop(0, CB // 16)
        def _(i):
            sdst_v[pl.ds(i * 16, 16)] = dst_v[pl.ds(i * 16, 16)]
        pltpu.async_copy(rows_v, acc_sh.at[sdst_v], ssem, add=True)
        fire_idx(lax.rem(j + 3, NCHUNK), src_v, dst_v, isem)

    for j in range(3):
        fire_idx(j, slots[j][0], slots[j][1], slots[j][4])

    for rows_v in (rows_a, rows_b, rows_c):
        @pl.loop(0, CB)
        def _(i):
            @pl.loop(0, D // 16)
            def _(j):
                rows_v[i, pl.ds(j * 16, 16)] = jnp.zeros((16,), jnp.float32)

    @pl.loop(0, RPT // CB)
    def _(k):
        pltpu.sync_copy(rows_c, acc_sh.at[pl.ds(s * RPT + k * CB, CB)])

    @pl.when(s == 0)
    def _():
        pltpu.sync_copy(rows_c, deg_sh)

    @pl.loop(0, DR)
    def _(i):
        @pl.loop(0, D // 16)
        def _(j):
            deg_loc[i, pl.ds(j * 16, 16)] = jnp.zeros((16,), jnp.float32)

    @pl.loop(0, DR // 16)
    def _(i):
        ident[pl.ds(i * 16, 16)] = lax.iota(jnp.int32, 16) + i * 16

    for slot in slots:
        _, _, sdst_v, rows_v, _, _, ssem = slot

        @pl.loop(0, DR // 16)
        def _(i):
            sdst_v[pl.ds(i * 16, 16)] = ident[pl.ds(i * 16, 16)]
        pltpu.async_copy(rows_v, acc_sh.at[sdst_v], ssem, add=True)

    start(0, slots[0])
    start(1, slots[1])

    plsc.subcore_barrier()

    @pl.loop(0, (NCHUNK - 2) // 3)
    def _(t):
        for k in range(3):
            j = 3 * t + k
            finish(j, slots[k])
            start(j + 2, slots[(k + 2) % 3])

    finish(NCHUNK - 2, slots[0])
    finish(NCHUNK - 1, slots[1])
    for k in range(3):
        wait_idx(slots[k][0], slots[k][1], slots[k][4])
        wait_scatter(slots[k])

    pltpu.sync_copy(deg_loc, deg_sh.at[ident], add=True)

    plsc.subcore_barrier()
    row0 = s * RPT
    pltpu.sync_copy(acc_sh.at[pl.ds(row0, RPT)],
                    csum_hbm.at[pl.ds(c * NP + row0, RPT)])

    @pl.when(s < 5)
    def _():
        pltpu.sync_copy(deg_sh.at[pl.ds(s * 16, 16)],
                        deg_hbm.at[pl.ds(c * DR + s * 16, 16)])


def _sc_segment(h1p, edge_index):
    mesh = plsc.VectorSubcoreMesh(
        core_axis_name="c", subcore_axis_name="s",
        num_cores=NC, num_subcores=NS)
    return pl.kernel(
        _sc_body,
        out_type=(jax.ShapeDtypeStruct((2 * NP, D), jnp.float32),
                  jax.ShapeDtypeStruct((2 * DR, D), jnp.float32)),
        mesh=mesh,
        compiler_params=pltpu.CompilerParams(needs_layout_passes=False),
        scratch_types=[
            pltpu.VMEM((CB,), jnp.int32),
            pltpu.VMEM((CB,), jnp.int32),
            pltpu.VMEM((CB,), jnp.int32),
            pltpu.VMEM((CB,), jnp.int32),
            pltpu.VMEM((CB,), jnp.int32),
            pltpu.VMEM((CB,), jnp.int32),
            pltpu.VMEM((CB,), jnp.int32),
            pltpu.VMEM((CB,), jnp.int32),
            pltpu.VMEM((CB,), jnp.int32),
            pltpu.VMEM((CB, D), jnp.float32),
            pltpu.VMEM((CB, D), jnp.float32),
            pltpu.VMEM((CB, D), jnp.float32),
            pltpu.VMEM((DR, D), jnp.float32),
            pltpu.VMEM((DR,), jnp.int32),
            pltpu.VMEM_SHARED((NP, D), jnp.float32),
            pltpu.VMEM_SHARED((DR, D), jnp.float32),
            pltpu.SemaphoreType.DMA,
            pltpu.SemaphoreType.DMA,
            pltpu.SemaphoreType.DMA,
            pltpu.SemaphoreType.DMA,
            pltpu.SemaphoreType.DMA,
            pltpu.SemaphoreType.DMA,
            pltpu.SemaphoreType.DMA,
            pltpu.SemaphoreType.DMA,
            pltpu.SemaphoreType.DMA,
        ],
    )(h1p, edge_index.reshape(2 * E))


def _epilogue_body(h_ref, norm_ref, csum_ref, deg_ref, gamma_ref, beta_ref,
                   out_ref):
    norm = norm_ref[...]
    h1 = h_ref[...] * norm
    rdeg = pl.reciprocal(
        jnp.maximum(deg_ref[:N, 0:1] + deg_ref[:N, 1:2], 1.0), approx=False)
    cmean = (csum_ref[:N, :] + csum_ref[NP:NP + N, :]) * rdeg
    ssq = (jnp.sum(h1 * h1, axis=1, keepdims=True)
           + jnp.sum(cmean * cmean, axis=1, keepdims=True))
    scale = norm * jax.lax.rsqrt(jnp.maximum(ssq, 1e-24))
    h2a = h1 * scale
    h2b = cmean * scale
    for half, h2 in enumerate((h2a, h2b)):
        mean = jnp.mean(h2, axis=0, keepdims=True)
        var = jnp.maximum(
            jnp.mean(h2 * h2, axis=0, keepdims=True) - mean * mean, 0.0)
        inv = jax.lax.rsqrt(var + 1e-5)
        a = inv * gamma_ref[:, half * D:(half + 1) * D]
        b = beta_ref[:, half * D:(half + 1) * D] - mean * a
        out_ref[:, half * D:(half + 1) * D] = h2 * a + b


def _epilogue(h, norm, csum, deg, gamma, beta):
    return pl.pallas_call(
        _epilogue_body,
        out_shape=jax.ShapeDtypeStruct((N, 2 * D), jnp.float32),
    )(h, norm, csum, jnp.transpose(deg.reshape(2, NP)),
      gamma.reshape(1, 2 * D), beta.reshape(1, 2 * D))


def kernel(h, edge_index, norm, gamma, beta):
    h1p = _prologue(h, norm)
    csum, deg = _sc_segment(h1p, edge_index)
    return _epilogue(h, norm, csum, deg, gamma, beta)


# speedup vs baseline: 1.1569x; 1.0010x over previous
import jax
import jax.numpy as jnp
from jax import lax
from jax.experimental import pallas as pl
from jax.experimental.pallas import tpu as pltpu
from jax.experimental.pallas import tpu_sc as plsc

N = 10000
NP = 10240
E = 320000
D = 128
NC = 2
NS = 16
CB = 80
EPS = E // NC
EPT = EPS // NS
NCHUNK = EPT // CB
RPT = NP // NS
DR = NP // D


def _prologue_body(h_ref, norm_ref, out_ref):
    h1 = h_ref[...] * norm_ref[...]
    pad = jnp.zeros((NP - N, D), jnp.float32)
    out_ref[...] = jnp.concatenate([h1, pad], axis=0)


def _prologue(h, norm):
    return pl.pallas_call(
        _prologue_body,
        out_shape=jax.ShapeDtypeStruct((NP, D), jnp.float32),
    )(h, norm)


def _sc_body(h1p, edges_hbm, csum_hbm, deg_hbm,
             src_a, src_b, src_c, dst_a, dst_b, dst_c,
             sdst_a, sdst_b, sdst_c,
             rows_a, rows_b, rows_c,
             deg_loc, ident, acc_sh, deg_sh,
             isem_a, isem_b, isem_c, gsem_a, gsem_b, gsem_c,
             ssem_a, ssem_b, ssem_c):
    c = lax.axis_index("c")
    s = lax.axis_index("s")
    ebase = c * EPS + s * EPT
    ones16 = jnp.ones((16,), jnp.float32)

    def fire_idx(j, src_v, dst_v, isem):
        off = ebase + j * CB
        pltpu.make_async_copy(
            edges_hbm.at[pl.ds(off, CB)], src_v, isem).start()
        pltpu.make_async_copy(
            edges_hbm.at[pl.ds(E + off, CB)], dst_v, isem).start()

    def wait_idx(src_v, dst_v, isem):
        pltpu.make_async_copy(
            edges_hbm.at[pl.ds(0, CB)], src_v, isem).wait()
        pltpu.make_async_copy(
            edges_hbm.at[pl.ds(0, CB)], dst_v, isem).wait()

    def fire_gather(src_v, rows_v, gsem):
        pltpu.make_async_copy(h1p.at[src_v], rows_v, gsem).start()

    def wait_gather(src_v, rows_v, gsem):
        pltpu.make_async_copy(h1p.at[src_v], rows_v, gsem).wait()

    def histogram(dst_v):
        for i in range(CB // 16):
            d16 = dst_v[pl.ds(i * 16, 16)]
            hi = lax.shift_right_logical(d16, 7)
            lo = lax.bitwise_and(d16, 127)
            counts, last = plsc.scan_count(d16)
            plsc.addupdate_scatter(
                deg_loc, [hi, lo], counts.astype(jnp.float32), mask=last)

    slots = ((src_a, dst_a, sdst_a, rows_a, isem_a, gsem_a, ssem_a),
             (src_b, dst_b, sdst_b, rows_b, isem_b, gsem_b, ssem_b),
             (src_c, dst_c, sdst_c, rows_c, isem_c, gsem_c, ssem_c))

    def wait_scatter(slot):
        _, _, sdst_v, rows_v, _, _, ssem = slot
        pltpu.make_async_copy(rows_v, acc_sh.at[sdst_v], ssem).wait()

    def start(j, slot):
        src_v, dst_v, sdst_v, rows_v, isem, gsem, ssem = slot
        wait_idx(src_v, dst_v, isem)
        wait_scatter(slot)
        fire_gather(src_v, rows_v, gsem)

    def finish(j, slot):
        src_v, dst_v, sdst_v, rows_v, isem, gsem, ssem = slot
        histogram(dst_v)
        wait_gather(src_v, rows_v, gsem)
        @pl.loop(0, CB // 16)
        def _(i):
            sdst_v[pl.ds(i * 16, 16)] = dst_v[pl.ds(i * 16, 16)]
        pltpu.async_copy(rows_v, acc_sh.at[sdst_v], ssem, add=True)
        fire_idx(lax.rem(j + 3, NCHUNK), src_v, dst_v, isem)

    for j in range(3):
        fire_idx(j, slots[j][0], slots[j][1], slots[j][4])

    for rows_v in (rows_a, rows_b, rows_c):
        @pl.loop(0, CB)
        def _(i):
            @pl.loop(0, D // 16)
            def _(j):
                rows_v[i, pl.ds(j * 16, 16)] = jnp.zeros((16,), jnp.float32)

    @pl.loop(0, RPT // CB)
    def _(k):
        pltpu.sync_copy(rows_c, acc_sh.at[pl.ds(s * RPT + k * CB, CB)])

    @pl.when(s == 0)
    def _():
        pltpu.sync_copy(rows_c, deg_sh)

    @pl.loop(0, DR)
    def _(i):
        @pl.loop(0, D // 16)
        def _(j):
            deg_loc[i, pl.ds(j * 16, 16)] = jnp.zeros((16,), jnp.float32)

    @pl.loop(0, DR // 16)
    def _(i):
        ident[pl.ds(i * 16, 16)] = lax.iota(jnp.int32, 16) + i * 16

    for slot in slots:
        _, _, sdst_v, rows_v, _, _, ssem = slot

        @pl.loop(0, DR // 16)
        def _(i):
            sdst_v[pl.ds(i * 16, 16)] = ident[pl.ds(i * 16, 16)]
        pltpu.async_copy(rows_v, acc_sh.at[sdst_v], ssem, add=True)

    start(0, slots[0])
    start(1, slots[1])

    plsc.subcore_barrier()

    @pl.loop(0, (NCHUNK - 2) // 3)
    def _(t):
        for k in range(3):
            j = 3 * t + k
            finish(j, slots[k])
            start(j + 2, slots[(k + 2) % 3])

    finish(NCHUNK - 2, slots[0])
    finish(NCHUNK - 1, slots[1])
    for k in range(3):
        wait_idx(slots[k][0], slots[k][1], slots[k][4])
        wait_scatter(slots[k])

    pltpu.sync_copy(deg_loc, deg_sh.at[ident], add=True)

    plsc.subcore_barrier()
    row0 = s * RPT
    pltpu.sync_copy(acc_sh.at[pl.ds(row0, RPT)],
                    csum_hbm.at[pl.ds(c * NP + row0, RPT)])

    @pl.when(s < 5)
    def _():
        pltpu.sync_copy(deg_sh.at[pl.ds(s * 16, 16)],
                        deg_hbm.at[pl.ds(c * DR + s * 16, 16)])


def _sc_segment(h1p, edge_index):
    mesh = plsc.VectorSubcoreMesh(
        core_axis_name="c", subcore_axis_name="s",
        num_cores=NC, num_subcores=NS)
    return pl.kernel(
        _sc_body,
        out_type=(jax.ShapeDtypeStruct((2 * NP, D), jnp.float32),
                  jax.ShapeDtypeStruct((2 * DR, D), jnp.float32)),
        mesh=mesh,
        compiler_params=pltpu.CompilerParams(needs_layout_passes=False),
        scratch_types=[
            pltpu.VMEM((CB,), jnp.int32),
            pltpu.VMEM((CB,), jnp.int32),
            pltpu.VMEM((CB,), jnp.int32),
            pltpu.VMEM((CB,), jnp.int32),
            pltpu.VMEM((CB,), jnp.int32),
            pltpu.VMEM((CB,), jnp.int32),
            pltpu.VMEM((CB,), jnp.int32),
            pltpu.VMEM((CB,), jnp.int32),
            pltpu.VMEM((CB,), jnp.int32),
            pltpu.VMEM((CB, D), jnp.float32),
            pltpu.VMEM((CB, D), jnp.float32),
            pltpu.VMEM((CB, D), jnp.float32),
            pltpu.VMEM((DR, D), jnp.float32),
            pltpu.VMEM((DR,), jnp.int32),
            pltpu.VMEM_SHARED((NP, D), jnp.float32),
            pltpu.VMEM_SHARED((DR, D), jnp.float32),
            pltpu.SemaphoreType.DMA,
            pltpu.SemaphoreType.DMA,
            pltpu.SemaphoreType.DMA,
            pltpu.SemaphoreType.DMA,
            pltpu.SemaphoreType.DMA,
            pltpu.SemaphoreType.DMA,
            pltpu.SemaphoreType.DMA,
            pltpu.SemaphoreType.DMA,
            pltpu.SemaphoreType.DMA,
        ],
    )(h1p, edge_index.reshape(2 * E))


def _epilogue_body(h_ref, norm_ref, csum_ref, deg_ref, gamma_ref, beta_ref,
                   out_ref):
    norm = norm_ref[...]
    h1 = h_ref[...] * norm
    rdeg = pl.reciprocal(
        jnp.maximum(deg_ref[:N, 0:1] + deg_ref[:N, 1:2], 1.0), approx=False)
    cmean = (csum_ref[:N, :] + csum_ref[NP:NP + N, :]) * rdeg
    ssq = (jnp.sum(h1 * h1, axis=1, keepdims=True)
           + jnp.sum(cmean * cmean, axis=1, keepdims=True))
    scale = norm * jax.lax.rsqrt(jnp.maximum(ssq, 1e-24))
    h2a = h1 * scale
    h2b = cmean * scale
    for half, h2 in enumerate((h2a, h2b)):
        mean = jnp.mean(h2, axis=0, keepdims=True)
        var = jnp.maximum(
            jnp.mean(h2 * h2, axis=0, keepdims=True) - mean * mean, 0.0)
        inv = jax.lax.rsqrt(var + 1e-5)
        a = inv * gamma_ref[:, half * D:(half + 1) * D]
        b = beta_ref[:, half * D:(half + 1) * D] - mean * a
        out_ref[:, half * D:(half + 1) * D] = h2 * a + b


def _epilogue(h, norm, csum, deg, gamma, beta):
    return pl.pallas_call(
        _epilogue_body,
        out_shape=jax.ShapeDtypeStruct((N, 2 * D), jnp.float32),
    )(h, norm, csum, jnp.transpose(deg.reshape(2, NP)),
      gamma.reshape(1, 2 * D), beta.reshape(1, 2 * D))


def kernel(h, edge_index, norm, gamma, beta):
    h1p = _prologue(h, norm)
    csum, deg = _sc_segment(h1p, edge_index)
    return _epilogue(h, norm, csum, deg, gamma, beta)
